# baseline (speedup 1.0000x reference)
import functools

import jax
import jax.numpy as jnp
from jax import lax
from jax.experimental import pallas as pl
from jax.experimental.pallas import tpu as pltpu
from jax.experimental.pallas import tpu_sc as plsc

N_NODES = 10000
NPAD = N_NODES + 8
EMB = 128
N_EDGES = 320000
NUM_LAYERS = 2
NC = 2
NS = 16
NW = NC * NS
K = 128
J = 80
EPT = J * K
E_PAD = NW * EPT
C_TOT = NW * J
JC0, JC1 = 120, 40
JD0, JD1 = 98, 62
JD0, JD1 = 136, 24


_MESH = plsc.VectorSubcoreMesh(
    core_axis_name="c", subcore_axis_name="s", num_cores=NC, num_subcores=NS
)
_SC_PARAMS = pltpu.CompilerParams(needs_layout_passes=False)


def _wid():
    return lax.axis_index("s") * NC + lax.axis_index("c")


DEG_PAD = 10240


@functools.partial(
    pl.kernel,
    out_type=jax.ShapeDtypeStruct((NC, DEG_PAD), jnp.float32),
    mesh=_MESH,
    compiler_params=_SC_PARAMS,
    scratch_types=[
        pltpu.VMEM((2, K), jnp.int32),
        pltpu.VMEM((2, K), jnp.int32),
        pltpu.VMEM((K,), jnp.float32),
        pltpu.VMEM_SHARED((10240,), jnp.float32),
        pltpu.SemaphoreType.DMA,
        pltpu.SemaphoreType.DMA,
    ],
)
def _deg_kernel(sd_hbm, zeros1_hbm, ones_hbm, degp_out,
                sd0, sd1, ones_v, acc, si0, si1):
    c = lax.axis_index("c")
    s = lax.axis_index("s")
    w = _wid()

    @pl.when(s == 0)
    def _():
        pltpu.sync_copy(zeros1_hbm, acc)

    pltpu.sync_copy(ones_hbm, ones_v)
    plsc.subcore_barrier()

    pltpu.async_copy(sd_hbm.at[w, 0], sd0, si0)
    pltpu.async_copy(sd_hbm.at[w, 1], sd1, si1)

    def body(i, carry):
        g = i * 2
        pltpu.make_async_copy(sd_hbm.at[w, g], sd0, si0).wait()
        pltpu.sync_copy(ones_v, acc.at[sd0.at[1]], add=True)

        @pl.when(g + 2 < J)
        def _():
            pltpu.async_copy(sd_hbm.at[w, g + 2], sd0, si0)

        pltpu.make_async_copy(sd_hbm.at[w, g + 1], sd1, si1).wait()
        pltpu.sync_copy(ones_v, acc.at[sd1.at[1]], add=True)

        @pl.when(g + 3 < J)
        def _():
            pltpu.async_copy(sd_hbm.at[w, g + 3], sd1, si1)

        return carry

    lax.fori_loop(0, J // 2, body, 0)
    plsc.subcore_barrier()

    @pl.when(s == 0)
    def _():
        pltpu.sync_copy(acc, degp_out.at[c])


@functools.partial(
    pl.kernel,
    out_type=jax.ShapeDtypeStruct((NC, N_NODES, EMB), jnp.float32),
    mesh=_MESH,
    compiler_params=_SC_PARAMS,
    scratch_types=[
        pltpu.VMEM((2, K), jnp.int32),
        pltpu.VMEM((2, K), jnp.int32),
        pltpu.VMEM((K, EMB), jnp.float32),
        pltpu.VMEM((K, EMB), jnp.float32),
        pltpu.VMEM_SHARED((NPAD, EMB), jnp.float32),
        pltpu.SemaphoreType.DMA,
        pltpu.SemaphoreType.DMA,
        pltpu.SemaphoreType.DMA,
        pltpu.SemaphoreType.DMA,
    ],
)
def _layer_kernel(y_hbm, sd_hbm, zeros2_hbm, part_out,
                  sd0, sd1, rows0, rows1, acc, si0, si1, sr0, sr1):
    c = lax.axis_index("c")
    s = lax.axis_index("s")

    @pl.when(s == 0)
    def _():
        pltpu.sync_copy(zeros2_hbm, acc)

    plsc.subcore_barrier()

    def run(base, jmy):
        pltpu.async_copy(sd_hbm.at[base + 0], sd0, si0)
        pltpu.async_copy(sd_hbm.at[base + 1], sd1, si1)
        pltpu.make_async_copy(sd_hbm.at[base + 0], sd0, si0).wait()
        pltpu.async_copy(y_hbm.at[sd0.at[0]], rows0, sr0)

        def body(i, carry):
            g = i * 2
            pltpu.make_async_copy(sd_hbm.at[base + g + 1], sd1, si1).wait()
            pltpu.async_copy(y_hbm.at[sd1.at[0]], rows1, sr1)

            pltpu.make_async_copy(y_hbm.at[sd0.at[0]], rows0, sr0).wait()
            pltpu.sync_copy(rows0, acc.at[sd0.at[1]], add=True)

            @pl.when(g + 2 < jmy)
            def _():
                pltpu.async_copy(sd_hbm.at[base + g + 2], sd0, si0)

            pltpu.make_async_copy(y_hbm.at[sd1.at[0]], rows1, sr1).wait()
            pltpu.sync_copy(rows1, acc.at[sd1.at[1]], add=True)

            @pl.when(g + 2 < jmy)
            def _():
                pltpu.make_async_copy(sd_hbm.at[base + g + 2], sd0, si0).wait()
                pltpu.async_copy(y_hbm.at[sd0.at[0]], rows0, sr0)

            @pl.when(g + 3 < jmy)
            def _():
                pltpu.async_copy(sd_hbm.at[base + g + 3], sd1, si1)

            return carry

        lax.fori_loop(0, jmy // 2, body, 0)

    @pl.when(c == 0)
    def _():
        run(s * JC0, JC0)

    @pl.when(c == 1)
    def _():
        run(NS * JC0 + s * JC1, JC1)

    plsc.subcore_barrier()

    @pl.when(s < 10)
    def _():
        pltpu.sync_copy(acc.at[pl.ds(s * 1000, 1000)],
                        part_out.at[c, pl.ds(s * 1000, 1000)])


GD = K // 16


TBL_PAD = 10048
HK = K // 2


@functools.partial(
    pl.kernel,
    out_type=jax.ShapeDtypeStruct((C_TOT, K), jnp.float32),
    mesh=_MESH,
    compiler_params=_SC_PARAMS,
    scratch_types=[
        pltpu.VMEM((2, K), jnp.int32),
        pltpu.VMEM((2, K), jnp.int32),
        pltpu.VMEM((K // 2, EMB), jnp.float32),
        pltpu.VMEM((K // 2, EMB), jnp.float32),
        pltpu.VMEM((K // 2, EMB), jnp.float32),
        pltpu.VMEM((K // 2, EMB), jnp.float32),
        pltpu.VMEM((8, K), jnp.float32),
        pltpu.VMEM((8, K), jnp.float32),
        pltpu.VMEM_SHARED((TBL_PAD, EMB), jnp.float32),
        pltpu.SemaphoreType.DMA,
        pltpu.SemaphoreType.DMA,
        pltpu.SemaphoreType.DMA,
        pltpu.SemaphoreType.DMA,
        pltpu.SemaphoreType.DMA,
        pltpu.SemaphoreType.DMA,
    ],
)
def _decode_kernel(outf_hbm, sd_hbm, dots_out,
                   sd0, sd1, sA0, dA0, sA1, dA1, db0, db1, table,
                   si0, si1, r0, r1, sw0, sw1):
    c = lax.axis_index("c")
    s = lax.axis_index("s")
    lane = lax.iota(jnp.int32, 16)

    @pl.when(jnp.logical_and(c == 1, s == 0))
    def _():
        pltpu.sync_copy(outf_hbm, table)

    plsc.subcore_barrier()
    base = jnp.where(c == 0, s * JD0, NS * JD0 + s * JD1)
    jmy = jnp.where(c == 0, JD0, JD1)

    def gather(sdb, half, sb, dbf, sem):
        @pl.when(c == 0)
        def _():
            pltpu.async_copy(
                outf_hbm.at[sdb.at[0, pl.ds(half * HK, HK)]], sb, sem)
            pltpu.async_copy(
                outf_hbm.at[sdb.at[1, pl.ds(half * HK, HK)]], dbf, sem)

        @pl.when(c == 1)
        def _():
            pltpu.async_copy(
                table.at[sdb.at[0, pl.ds(half * HK, HK)]], sb, sem)
            pltpu.async_copy(
                table.at[sdb.at[1, pl.ds(half * HK, HK)]], dbf, sem)

    def gwait(sdb, half, sb, dbf, sem):
        pltpu.make_async_copy(
            outf_hbm.at[sdb.at[0, pl.ds(half * HK, HK)]], sb, sem).wait()
        pltpu.make_async_copy(
            outf_hbm.at[sdb.at[1, pl.ds(half * HK, HK)]], dbf, sem).wait()

    def compute(dbuf, half, sb, dbf):
        def group(g, c2):
            gbase = g * 16
            dots = []
            for t in range(16):
                e = gbase + t
                prod = sb[e, pl.ds(0, 16)] * dbf[e, pl.ds(0, 16)]
                for q in range(1, EMB // 16):
                    prod = prod + (sb[e, pl.ds(q * 16, 16)]
                                   * dbf[e, pl.ds(q * 16, 16)])
                dots.append(jnp.sum(prod))
            accvec = jnp.zeros((16,), jnp.float32)
            for t in range(16):
                accvec = jnp.where(lane == t, dots[t], accvec)
            dbuf[0, pl.ds(half * HK + gbase, 16)] = accvec
            return c2

        lax.fori_loop(0, HK // 16, group, 0)

    pltpu.async_copy(sd_hbm.at[base + 0], sd0, si0)
    pltpu.async_copy(sd_hbm.at[base + 1], sd1, si1)
    pltpu.make_async_copy(sd_hbm.at[base + 0], sd0, si0).wait()
    gather(sd0, 0, sA0, dA0, r0)

    def body(i, carry):
        g = i * 2
        pltpu.make_async_copy(sd_hbm.at[base + g + 1], sd1, si1).wait()
        gather(sd0, 1, sA1, dA1, r1)
        gwait(sd0, 0, sA0, dA0, r0)

        @pl.when(i > 0)
        def _():
            pltpu.make_async_copy(
                db0.at[0], dots_out.at[base + g - 2], sw0).wait()

        compute(db0, 0, sA0, dA0)
        gather(sd1, 0, sA0, dA0, r0)
        gwait(sd0, 1, sA1, dA1, r1)
        compute(db0, 1, sA1, dA1)
        pltpu.async_copy(db0.at[0], dots_out.at[base + g], sw0)

        @pl.when(g + 2 < jmy)
        def _():
            pltpu.async_copy(sd_hbm.at[base + g + 2], sd0, si0)

        gather(sd1, 1, sA1, dA1, r1)
        gwait(sd1, 0, sA0, dA0, r0)

        @pl.when(i > 0)
        def _():
            pltpu.make_async_copy(
                db1.at[0], dots_out.at[base + g - 1], sw1).wait()

        compute(db1, 0, sA0, dA0)

        @pl.when(g + 2 < jmy)
        def _():
            pltpu.make_async_copy(sd_hbm.at[base + g + 2], sd0, si0).wait()
            gather(sd0, 0, sA0, dA0, r0)

        gwait(sd1, 1, sA1, dA1, r1)
        compute(db1, 1, sA1, dA1)
        pltpu.async_copy(db1.at[0], dots_out.at[base + g + 1], sw1)

        @pl.when(g + 3 < jmy)
        def _():
            pltpu.async_copy(sd_hbm.at[base + g + 3], sd1, si1)

        return carry

    lax.fori_loop(0, jmy // 2, body, 0)
    pltpu.make_async_copy(db0.at[0], dots_out.at[base + jmy - 2], sw0).wait()
    pltpu.make_async_copy(db1.at[0], dots_out.at[base + jmy - 1], sw1).wait()


def _tc1_body(degt_ref, emb_ref, y0_ref, dinv_ref):
    deg = degt_ref[:, 0:1] + degt_ref[:, 1:2]
    dinv = jnp.where(deg > 0, lax.rsqrt(jnp.maximum(deg, 1.0)), 0.0)
    dinv_ref[...] = dinv
    y0_ref[...] = emb_ref[...] * dinv


def _tc2_body(part_ref, dinv_ref, x1_ref, y1_ref):
    dinv = dinv_ref[...]
    x1 = (part_ref[0] + part_ref[1]) * dinv
    x1_ref[...] = x1
    y1_ref[...] = x1 * dinv


def _tc3_body(part_ref, dinv_ref, emb_ref, x1_ref, outf_ref):
    alpha = 1.0 / (NUM_LAYERS + 1)
    x2 = (part_ref[0] + part_ref[1]) * dinv_ref[...]
    outf_ref[...] = alpha * (emb_ref[...] + x1_ref[...] + x2)


_tc1 = pl.pallas_call(
    _tc1_body,
    out_shape=[jax.ShapeDtypeStruct((N_NODES, EMB), jnp.float32),
               jax.ShapeDtypeStruct((N_NODES, 1), jnp.float32)],
)
_tc2 = pl.pallas_call(
    _tc2_body,
    out_shape=[jax.ShapeDtypeStruct((N_NODES, EMB), jnp.float32),
               jax.ShapeDtypeStruct((N_NODES, EMB), jnp.float32)],
)
_tc3 = pl.pallas_call(
    _tc3_body,
    out_shape=jax.ShapeDtypeStruct((N_NODES, EMB), jnp.float32),
)


def _pack_edges(a, b, fill_a, fill_b):
    pa = jnp.full((E_PAD - N_EDGES,), fill_a, jnp.int32)
    pb = jnp.full((E_PAD - N_EDGES,), fill_b, jnp.int32)
    aa = jnp.concatenate([a, pa]).reshape(NW, J, 1, K)
    bb = jnp.concatenate([b, pb]).reshape(NW, J, 1, K)
    return jnp.concatenate([aa, bb], axis=2)


def kernel(edge_index, edge_label_index, emb):
    sd = _pack_edges(edge_index[0], edge_index[1], 0, N_NODES)
    sd_lbl = _pack_edges(edge_label_index[0], edge_label_index[1], 0, 0)
    zeros1 = jnp.zeros((10240,), jnp.float32)
    zeros2 = jnp.zeros((NPAD, EMB), jnp.float32)
    ones_k = jnp.ones((K,), jnp.float32)

    degp = _deg_kernel(sd, zeros1, ones_k)
    degt = degp.T[:N_NODES]
    y0, dinv = _tc1(degt, emb)

    sdf = sd.reshape(C_TOT, 2, K)
    part1 = _layer_kernel(y0, sdf, zeros2)
    x1, y1 = _tc2(part1, dinv)

    part2 = _layer_kernel(y1, sdf, zeros2)
    outf = _tc3(part2, dinv, emb, x1)

    outf_p = jnp.concatenate(
        [outf, jnp.zeros((TBL_PAD - N_NODES, EMB), jnp.float32)])
    sdlf = sd_lbl.reshape(C_TOT, 2, K)
    dots = _decode_kernel(outf_p, sdlf)
    return dots.reshape(E_PAD)[:N_EDGES]


# speedup vs baseline: 1.2123x; 1.2123x over previous
import functools

import jax
import jax.numpy as jnp
from jax import lax
from jax.experimental import pallas as pl
from jax.experimental.pallas import tpu as pltpu
from jax.experimental.pallas import tpu_sc as plsc

N_NODES = 10000
NPAD = N_NODES + 8
EMB = 128
N_EDGES = 320000
NUM_LAYERS = 2
NC = 2
NS = 16
NW = NC * NS
K = 128
J = 80
EPT = J * K
E_PAD = NW * EPT
C_TOT = NW * J
JC0, JC1 = 120, 40
JD0, JD1 = 136, 24


_MESH = plsc.VectorSubcoreMesh(
    core_axis_name="c", subcore_axis_name="s", num_cores=NC, num_subcores=NS
)
_SC_PARAMS = pltpu.CompilerParams(needs_layout_passes=False)


def _wid():
    return lax.axis_index("s") * NC + lax.axis_index("c")


DEG_PAD = 10240


@functools.partial(
    pl.kernel,
    out_type=jax.ShapeDtypeStruct((NC, DEG_PAD), jnp.float32),
    mesh=_MESH,
    compiler_params=_SC_PARAMS,
    scratch_types=[
        pltpu.VMEM((2, K), jnp.int32),
        pltpu.VMEM((2, K), jnp.int32),
        pltpu.VMEM((K,), jnp.float32),
        pltpu.VMEM_SHARED((10240,), jnp.float32),
        pltpu.SemaphoreType.DMA,
        pltpu.SemaphoreType.DMA,
    ],
)
def _deg_kernel(sd_hbm, zeros1_hbm, ones_hbm, degp_out,
                sd0, sd1, ones_v, acc, si0, si1):
    c = lax.axis_index("c")
    s = lax.axis_index("s")
    w = _wid()

    @pl.when(s == 0)
    def _():
        pltpu.sync_copy(zeros1_hbm, acc)

    pltpu.sync_copy(ones_hbm, ones_v)
    plsc.subcore_barrier()

    pltpu.async_copy(sd_hbm.at[w, 0], sd0, si0)
    pltpu.async_copy(sd_hbm.at[w, 1], sd1, si1)

    def body(i, carry):
        g = i * 2
        pltpu.make_async_copy(sd_hbm.at[w, g], sd0, si0).wait()
        pltpu.sync_copy(ones_v, acc.at[sd0.at[1]], add=True)

        @pl.when(g + 2 < J)
        def _():
            pltpu.async_copy(sd_hbm.at[w, g + 2], sd0, si0)

        pltpu.make_async_copy(sd_hbm.at[w, g + 1], sd1, si1).wait()
        pltpu.sync_copy(ones_v, acc.at[sd1.at[1]], add=True)

        @pl.when(g + 3 < J)
        def _():
            pltpu.async_copy(sd_hbm.at[w, g + 3], sd1, si1)

        return carry

    lax.fori_loop(0, J // 2, body, 0)
    plsc.subcore_barrier()

    @pl.when(s == 0)
    def _():
        pltpu.sync_copy(acc, degp_out.at[c])


@functools.partial(
    pl.kernel,
    out_type=jax.ShapeDtypeStruct((NC, N_NODES, EMB), jnp.float32),
    mesh=_MESH,
    compiler_params=_SC_PARAMS,
    scratch_types=[
        pltpu.VMEM((2, K), jnp.int32),
        pltpu.VMEM((2, K), jnp.int32),
        pltpu.VMEM((K, EMB), jnp.float32),
        pltpu.VMEM((K, EMB), jnp.float32),
        pltpu.VMEM_SHARED((NPAD, EMB), jnp.float32),
        pltpu.SemaphoreType.DMA,
        pltpu.SemaphoreType.DMA,
        pltpu.SemaphoreType.DMA,
        pltpu.SemaphoreType.DMA,
    ],
)
def _layer_kernel(y_hbm, sd_hbm, zeros2_hbm, part_out,
                  sd0, sd1, rows0, rows1, acc, si0, si1, sr0, sr1):
    c = lax.axis_index("c")
    s = lax.axis_index("s")

    @pl.when(s == 0)
    def _():
        pltpu.sync_copy(zeros2_hbm, acc)

    plsc.subcore_barrier()

    def run(base, jmy):
        pltpu.async_copy(sd_hbm.at[base + 0], sd0, si0)
        pltpu.async_copy(sd_hbm.at[base + 1], sd1, si1)
        pltpu.make_async_copy(sd_hbm.at[base + 0], sd0, si0).wait()
        pltpu.async_copy(y_hbm.at[sd0.at[0]], rows0, sr0)

        def body(i, carry):
            g = i * 2
            pltpu.make_async_copy(sd_hbm.at[base + g + 1], sd1, si1).wait()
            pltpu.async_copy(y_hbm.at[sd1.at[0]], rows1, sr1)

            pltpu.make_async_copy(y_hbm.at[sd0.at[0]], rows0, sr0).wait()
            pltpu.sync_copy(rows0, acc.at[sd0.at[1]], add=True)

            @pl.when(g + 2 < jmy)
            def _():
                pltpu.async_copy(sd_hbm.at[base + g + 2], sd0, si0)

            pltpu.make_async_copy(y_hbm.at[sd1.at[0]], rows1, sr1).wait()
            pltpu.sync_copy(rows1, acc.at[sd1.at[1]], add=True)

            @pl.when(g + 2 < jmy)
            def _():
                pltpu.make_async_copy(sd_hbm.at[base + g + 2], sd0, si0).wait()
                pltpu.async_copy(y_hbm.at[sd0.at[0]], rows0, sr0)

            @pl.when(g + 3 < jmy)
            def _():
                pltpu.async_copy(sd_hbm.at[base + g + 3], sd1, si1)

            return carry

        lax.fori_loop(0, jmy // 2, body, 0)

    @pl.when(c == 0)
    def _():
        run(s * JC0, JC0)

    @pl.when(c == 1)
    def _():
        run(NS * JC0 + s * JC1, JC1)

    plsc.subcore_barrier()

    @pl.when(s < 10)
    def _():
        pltpu.sync_copy(acc.at[pl.ds(s * 1000, 1000)],
                        part_out.at[c, pl.ds(s * 1000, 1000)])


GD = K // 16


TBL_PAD = 10048
HK = K // 2


@functools.partial(
    pl.kernel,
    out_type=jax.ShapeDtypeStruct((NW, J, K), jnp.float32),
    mesh=_MESH,
    compiler_params=_SC_PARAMS,
    scratch_types=[
        pltpu.VMEM((2, K), jnp.int32),
        pltpu.VMEM((2, K), jnp.int32),
        pltpu.VMEM((K // 2, EMB), jnp.float32),
        pltpu.VMEM((K // 2, EMB), jnp.float32),
        pltpu.VMEM((K // 2, EMB), jnp.float32),
        pltpu.VMEM((K // 2, EMB), jnp.float32),
        pltpu.VMEM((8, K), jnp.float32),
        pltpu.VMEM((8, K), jnp.float32),
        pltpu.VMEM_SHARED((TBL_PAD, EMB), jnp.float32),
        pltpu.SemaphoreType.DMA,
        pltpu.SemaphoreType.DMA,
        pltpu.SemaphoreType.DMA,
        pltpu.SemaphoreType.DMA,
        pltpu.SemaphoreType.DMA,
        pltpu.SemaphoreType.DMA,
    ],
)
def _decode_kernel(outf_hbm, sd_hbm, dots_out,
                   sd0, sd1, sA0, dA0, sA1, dA1, db0, db1, table,
                   si0, si1, r0, r1, sw0, sw1):
    s = lax.axis_index("s")
    w = _wid()
    lane = lax.iota(jnp.int32, 16)

    @pl.when(s == 0)
    def _():
        pltpu.sync_copy(outf_hbm, table)

    pltpu.async_copy(sd_hbm.at[w, 0], sd0, si0)
    pltpu.async_copy(sd_hbm.at[w, 1], sd1, si1)
    plsc.subcore_barrier()
    pltpu.make_async_copy(sd_hbm.at[w, 0], sd0, si0).wait()

    def gather(sdb, half, sb, dbf, sem):
        pltpu.async_copy(table.at[sdb.at[0, pl.ds(half * HK, HK)]], sb, sem)
        pltpu.async_copy(table.at[sdb.at[1, pl.ds(half * HK, HK)]], dbf, sem)

    def gwait(sdb, half, sb, dbf, sem):
        pltpu.make_async_copy(
            table.at[sdb.at[0, pl.ds(half * HK, HK)]], sb, sem).wait()
        pltpu.make_async_copy(
            table.at[sdb.at[1, pl.ds(half * HK, HK)]], dbf, sem).wait()

    def compute(dbuf, half, sb, dbf):
        def group(g, c2):
            gbase = g * 16
            dots = []
            for t in range(16):
                e = gbase + t
                prod = sb[e, pl.ds(0, 16)] * dbf[e, pl.ds(0, 16)]
                for q in range(1, EMB // 16):
                    prod = prod + (sb[e, pl.ds(q * 16, 16)]
                                   * dbf[e, pl.ds(q * 16, 16)])
                dots.append(jnp.sum(prod))
            accvec = jnp.zeros((16,), jnp.float32)
            for t in range(16):
                accvec = jnp.where(lane == t, dots[t], accvec)
            dbuf[0, pl.ds(half * HK + gbase, 16)] = accvec
            return c2

        lax.fori_loop(0, HK // 16, group, 0)

    gather(sd0, 0, sA0, dA0, r0)

    def body(i, carry):
        g = i * 2
        pltpu.make_async_copy(sd_hbm.at[w, g + 1], sd1, si1).wait()
        gather(sd0, 1, sA1, dA1, r1)
        gwait(sd0, 0, sA0, dA0, r0)

        @pl.when(i > 0)
        def _():
            pltpu.make_async_copy(db0.at[0], dots_out.at[w, g - 2], sw0).wait()

        compute(db0, 0, sA0, dA0)
        gather(sd1, 0, sA0, dA0, r0)
        gwait(sd0, 1, sA1, dA1, r1)
        compute(db0, 1, sA1, dA1)
        pltpu.async_copy(db0.at[0], dots_out.at[w, g], sw0)

        @pl.when(g + 2 < J)
        def _():
            pltpu.async_copy(sd_hbm.at[w, g + 2], sd0, si0)

        gather(sd1, 1, sA1, dA1, r1)
        gwait(sd1, 0, sA0, dA0, r0)

        @pl.when(i > 0)
        def _():
            pltpu.make_async_copy(db1.at[0], dots_out.at[w, g - 1], sw1).wait()

        compute(db1, 0, sA0, dA0)

        @pl.when(g + 2 < J)
        def _():
            pltpu.make_async_copy(sd_hbm.at[w, g + 2], sd0, si0).wait()
            gather(sd0, 0, sA0, dA0, r0)

        gwait(sd1, 1, sA1, dA1, r1)
        compute(db1, 1, sA1, dA1)
        pltpu.async_copy(db1.at[0], dots_out.at[w, g + 1], sw1)

        @pl.when(g + 3 < J)
        def _():
            pltpu.async_copy(sd_hbm.at[w, g + 3], sd1, si1)

        return carry

    lax.fori_loop(0, J // 2, body, 0)
    pltpu.make_async_copy(db0.at[0], dots_out.at[w, J - 2], sw0).wait()
    pltpu.make_async_copy(db1.at[0], dots_out.at[w, J - 1], sw1).wait()


def _tc1_body(degt_ref, emb_ref, y0_ref, dinv_ref):
    deg = degt_ref[:, 0:1] + degt_ref[:, 1:2]
    dinv = jnp.where(deg > 0, lax.rsqrt(jnp.maximum(deg, 1.0)), 0.0)
    dinv_ref[...] = dinv
    y0_ref[...] = emb_ref[...] * dinv


def _tc2_body(part_ref, dinv_ref, x1_ref, y1_ref):
    dinv = dinv_ref[...]
    x1 = (part_ref[0] + part_ref[1]) * dinv
    x1_ref[...] = x1
    y1_ref[...] = x1 * dinv


def _tc3_body(part_ref, dinv_ref, emb_ref, x1_ref, outf_ref):
    alpha = 1.0 / (NUM_LAYERS + 1)
    x2 = (part_ref[0] + part_ref[1]) * dinv_ref[...]
    outf_ref[...] = alpha * (emb_ref[...] + x1_ref[...] + x2)


_tc1 = pl.pallas_call(
    _tc1_body,
    out_shape=[jax.ShapeDtypeStruct((N_NODES, EMB), jnp.float32),
               jax.ShapeDtypeStruct((N_NODES, 1), jnp.float32)],
)
_tc2 = pl.pallas_call(
    _tc2_body,
    out_shape=[jax.ShapeDtypeStruct((N_NODES, EMB), jnp.float32),
               jax.ShapeDtypeStruct((N_NODES, EMB), jnp.float32)],
)
_tc3 = pl.pallas_call(
    _tc3_body,
    out_shape=jax.ShapeDtypeStruct((N_NODES, EMB), jnp.float32),
)


def _pack_edges(a, b, fill_a, fill_b):
    pa = jnp.full((E_PAD - N_EDGES,), fill_a, jnp.int32)
    pb = jnp.full((E_PAD - N_EDGES,), fill_b, jnp.int32)
    aa = jnp.concatenate([a, pa]).reshape(NW, J, 1, K)
    bb = jnp.concatenate([b, pb]).reshape(NW, J, 1, K)
    return jnp.concatenate([aa, bb], axis=2)


def kernel(edge_index, edge_label_index, emb):
    sd = _pack_edges(edge_index[0], edge_index[1], 0, N_NODES)
    sd_lbl = _pack_edges(edge_label_index[0], edge_label_index[1], 0, 0)
    zeros1 = jnp.zeros((10240,), jnp.float32)
    zeros2 = jnp.zeros((NPAD, EMB), jnp.float32)
    ones_k = jnp.ones((K,), jnp.float32)

    degp = _deg_kernel(sd, zeros1, ones_k)
    degt = degp.T[:N_NODES]
    y0, dinv = _tc1(degt, emb)

    sdf = sd.reshape(C_TOT, 2, K)
    part1 = _layer_kernel(y0, sdf, zeros2)
    x1, y1 = _tc2(part1, dinv)

    part2 = _layer_kernel(y1, sdf, zeros2)
    outf = _tc3(part2, dinv, emb, x1)

    outf_p = jnp.concatenate(
        [outf, jnp.zeros((TBL_PAD - N_NODES, EMB), jnp.float32)])
    dots = _decode_kernel(outf_p, sd_lbl)
    return dots.reshape(E_PAD)[:N_EDGES]
